# Initial kernel scaffold; baseline (speedup 1.0000x reference)
import jax
import jax.numpy as jnp
from jax.experimental import pallas as pl


def kernel(user_indices, item_indices, user_table, item_table, adj_rows, adj_cols):
    raise NotImplementedError("write your pallas kernel here")


# speedup vs baseline: 5.3556x; 5.3556x over previous
import functools

import jax
import jax.numpy as jnp
from jax import lax
from jax.experimental import pallas as pl
from jax.experimental.pallas import tpu as pltpu
from jax.experimental.pallas import tpu_sc as plsc

N_ROWS = 50000
N_EDGES_TOTAL = 800000
DIM = 64
BATCH_SIZE = 4096

NUM_SC = 2
NUM_TILES = 16
CHUNK = 128
N_CHUNKS = N_EDGES_TOTAL // CHUNK
HALF = N_ROWS // 2
ACC_ROWS = HALF + 8
TILE_ACC = ACC_ROWS // NUM_TILES
WB_CHUNK = 40
N_WB = HALF // WB_CHUNK

_SC_MESH = plsc.VectorSubcoreMesh(core_axis_name="c", subcore_axis_name="s")


def _remap_dst(idx_d, idx_a, lo):
    for i in range(CHUNK // 16):
        dv = idx_d[pl.ds(i * 16, 16)]
        ok = (dv >= lo) & (dv < lo + HALF)
        idx_a[pl.ds(i * 16, 16)] = jnp.where(ok, dv - lo, HALF)


def _zero_vmem_2d(buf, n_rows):
    def body(r, _):
        for j in range(DIM // 16):
            buf[r, pl.ds(j * 16, 16)] = jnp.zeros((16,), jnp.float32)
        return 0

    lax.fori_loop(0, n_rows, body, 0)


def _spmm_body(with_rowsum, src_hbm, dst_hbm, table_hbm, *rest):
    if with_rowsum:
        out_hbm, rs_hbm, idx_s, idx_d, idx_a, rows, zbuf, ones, acc, acc1, sem = rest
    else:
        out_hbm, idx_s, idx_d, idx_a, rows, zbuf, ones, acc, sem = rest
        rs_hbm = acc1 = None
    c = lax.axis_index("c")
    s = lax.axis_index("s")
    lo = c * HALF

    _zero_vmem_2d(zbuf, CHUNK)
    for j in range(CHUNK // 16):
        ones[pl.ds(j * 16, 16)] = jnp.ones((16,), jnp.float32)
    z0 = s * TILE_ACC
    n_full = TILE_ACC // CHUNK
    tail = TILE_ACC - n_full * CHUNK

    def zbody(j, _):
        pltpu.sync_copy(zbuf, acc.at[pl.ds(z0 + j * CHUNK, CHUNK), :])
        return 0

    lax.fori_loop(0, n_full, zbody, 0)
    pltpu.sync_copy(zbuf.at[pl.ds(0, tail), :], acc.at[pl.ds(z0 + n_full * CHUNK, tail), :])
    if with_rowsum:
        zcol = zbuf.at[0, pl.ds(0, 16)]
        n_z1_chunks = ACC_ROWS // 16
        n_z1 = n_z1_chunks // NUM_TILES + jnp.where(s < n_z1_chunks % NUM_TILES, 1, 0)

        def z1body(j, _):
            pltpu.sync_copy(zcol, acc1.at[pl.ds((s + j * NUM_TILES) * 16, 16)])
            return 0

        lax.fori_loop(0, n_z1, z1body, 0)
    plsc.subcore_barrier()

    n_my = N_CHUNKS // NUM_TILES + jnp.where(s < N_CHUNKS % NUM_TILES, 1, 0)

    def ebody(j, _):
        off = (s + j * NUM_TILES) * CHUNK
        pltpu.sync_copy(src_hbm.at[pl.ds(off, CHUNK)], idx_s)
        pltpu.sync_copy(dst_hbm.at[pl.ds(off, CHUNK)], idx_d)
        cp = pltpu.async_copy(table_hbm.at[idx_s], rows, sem)
        _remap_dst(idx_d, idx_a, lo)
        cp.wait()
        pltpu.sync_copy(rows, acc.at[idx_a], add=True)
        if with_rowsum:
            pltpu.sync_copy(ones, acc1.at[idx_a], add=True)
        return 0

    lax.fori_loop(0, n_my, ebody, 0)
    plsc.subcore_barrier()

    n_wb = N_WB // NUM_TILES + jnp.where(s < N_WB % NUM_TILES, 1, 0)

    def wbody(j, _):
        r0 = (s + j * NUM_TILES) * WB_CHUNK
        pltpu.sync_copy(acc.at[pl.ds(r0, WB_CHUNK), :], out_hbm.at[pl.ds(lo + r0, WB_CHUNK), :])
        if with_rowsum:
            pltpu.sync_copy(acc1.at[pl.ds(r0, WB_CHUNK)], rs_hbm.at[pl.ds(lo + r0, WB_CHUNK)])
        return 0

    lax.fori_loop(0, n_wb, wbody, 0)


def _make_spmm(with_rowsum):
    out_type = [jax.ShapeDtypeStruct((N_ROWS, DIM), jnp.float32)]
    scratch = [
        pltpu.VMEM((CHUNK,), jnp.int32),
        pltpu.VMEM((CHUNK,), jnp.int32),
        pltpu.VMEM((CHUNK,), jnp.int32),
        pltpu.VMEM((CHUNK, DIM), jnp.float32),
        pltpu.VMEM((CHUNK, DIM), jnp.float32),
        pltpu.VMEM((CHUNK,), jnp.float32),
        pltpu.VMEM_SHARED((ACC_ROWS, DIM), jnp.float32),
    ]
    if with_rowsum:
        out_type.append(jax.ShapeDtypeStruct((N_ROWS,), jnp.float32))
        scratch.append(pltpu.VMEM_SHARED((ACC_ROWS,), jnp.float32))
    scratch.append(pltpu.SemaphoreType.DMA)
    return pl.kernel(
        functools.partial(_spmm_body, with_rowsum),
        out_type=tuple(out_type) if with_rowsum else out_type[0],
        mesh=_SC_MESH,
        scratch_types=scratch,
        compiler_params=pltpu.CompilerParams(use_tc_tiling_on_sc=False),
        name="spmm_rowsum" if with_rowsum else "spmm",
    )


_spmm_first = _make_spmm(True)
_spmm = _make_spmm(False)


def _gather_body(t2, i2, rs, u_idx, i_idx, u_out, i_out, rsg_out, idx_v, rows_v, val_v, sem):
    c = lax.axis_index("c")
    s = lax.axis_index("s")
    base = (s * NUM_SC + c) * CHUNK

    pltpu.sync_copy(u_idx.at[pl.ds(base, CHUNK)], idx_v)
    pltpu.async_copy(t2.at[idx_v], rows_v, sem).wait()
    pltpu.sync_copy(rows_v, u_out.at[pl.ds(base, CHUNK), :])
    pltpu.async_copy(rs.at[idx_v], val_v, sem).wait()
    pltpu.sync_copy(val_v, rsg_out.at[pl.ds(base, CHUNK)])

    pltpu.sync_copy(i_idx.at[pl.ds(base, CHUNK)], idx_v)
    pltpu.async_copy(i2.at[idx_v], rows_v, sem).wait()
    pltpu.sync_copy(rows_v, i_out.at[pl.ds(base, CHUNK), :])


_gather = pl.kernel(
    _gather_body,
    out_type=(
        jax.ShapeDtypeStruct((BATCH_SIZE, DIM), jnp.float32),
        jax.ShapeDtypeStruct((BATCH_SIZE, DIM), jnp.float32),
        jax.ShapeDtypeStruct((BATCH_SIZE,), jnp.float32),
    ),
    mesh=_SC_MESH,
    scratch_types=[
        pltpu.VMEM((CHUNK,), jnp.int32),
        pltpu.VMEM((CHUNK, DIM), jnp.float32),
        pltpu.VMEM((CHUNK,), jnp.float32),
        pltpu.SemaphoreType.DMA,
    ],
    compiler_params=pltpu.CompilerParams(use_tc_tiling_on_sc=False),
    name="batch_gather",
)


_SCALE_BLK = 2000


def _scale_kernel(t_ref, rs_ref, o_ref):
    rs = rs_ref[...]
    inv = jnp.where(rs > 0, 1.0 / rs, 0.0)
    o_ref[...] = t_ref[...] * (inv * inv)


def _scale_rows(t, rs):
    grid = N_ROWS // _SCALE_BLK
    return pl.pallas_call(
        _scale_kernel,
        grid=(grid,),
        in_specs=[
            pl.BlockSpec((_SCALE_BLK, DIM), lambda i: (i, 0)),
            pl.BlockSpec((_SCALE_BLK, 1), lambda i: (i, 0)),
        ],
        out_specs=pl.BlockSpec((_SCALE_BLK, DIM), lambda i: (i, 0)),
        out_shape=jax.ShapeDtypeStruct((N_ROWS, DIM), jnp.float32),
    )(t, rs.reshape(N_ROWS, 1))


def _score_kernel(u_ref, i_ref, rs_ref, o_ref):
    dot = jnp.sum(u_ref[...] * i_ref[...], axis=1, keepdims=True)
    rs = rs_ref[...]
    inv = jnp.where(rs > 0, 1.0 / rs, 0.0)
    o_ref[...] = jax.nn.sigmoid(dot * inv)


def _score(u_rows, i_rows, rs_g):
    out = pl.pallas_call(
        _score_kernel,
        out_shape=jax.ShapeDtypeStruct((BATCH_SIZE, 1), jnp.float32),
    )(u_rows, i_rows, rs_g.reshape(BATCH_SIZE, 1))
    return out.reshape(BATCH_SIZE)


def kernel(user_indices, item_indices, user_table, item_table, adj_rows, adj_cols):
    del user_table
    t1, rowsum = _spmm_first(adj_cols, adj_rows, item_table)
    us1 = _scale_rows(t1, rowsum)
    i1 = _spmm(adj_rows, adj_cols, us1)
    t2 = _spmm(adj_cols, adj_rows, i1)
    us2 = _scale_rows(t2, rowsum)
    i2 = _spmm(adj_rows, adj_cols, us2)
    u_rows, i_rows, rs_g = _gather(t2, i2, rowsum, user_indices, item_indices)
    return _score(u_rows, i_rows, rs_g)


# speedup vs baseline: 13.3918x; 2.5005x over previous
import functools

import jax
import jax.numpy as jnp
from jax import lax
from jax.experimental import pallas as pl
from jax.experimental.pallas import tpu as pltpu
from jax.experimental.pallas import tpu_sc as plsc

N_ROWS = 50000
N_EDGES_TOTAL = 800000
DIM = 64
HDIM = DIM // 2
BATCH_SIZE = 4096

NUM_SC = 2
NUM_TILES = 16
CHUNK = 128
BLK = 5
N_IDX_ROWS = N_EDGES_TOTAL // CHUNK
N_BLOCKS = N_IDX_ROWS // BLK
ZERO_CHUNK = 125
TILE_ACC = N_ROWS // NUM_TILES
WB_CHUNK = 40
N_WB = N_ROWS // WB_CHUNK

_SC_MESH = plsc.VectorSubcoreMesh(core_axis_name="c", subcore_axis_name="s")
_SC_PARAMS = pltpu.CompilerParams(use_tc_tiling_on_sc=False)


def _spmm_body(with_rowsum, src_hbm, dst_hbm, table_hbm, *rest):
    if with_rowsum:
        (out_hbm, rs_hbm, idx_s, idx_d, rows, ones, acc, acc1,
         sem_g, sem_s, sem_1) = rest
    else:
        out_hbm, idx_s, idx_d, rows, ones, acc, sem_g, sem_s, sem_1 = rest
        rs_hbm = acc1 = None
    c = lax.axis_index("c")
    s = lax.axis_index("s")

    def zrow(r, _):
        for j in range(HDIM // 16):
            rows[0, r, pl.ds(j * 16, 16)] = jnp.zeros((16,), jnp.float32)
        return 0

    lax.fori_loop(0, CHUNK, zrow, 0)
    for j in range(CHUNK // 16):
        ones[pl.ds(j * 16, 16)] = jnp.ones((16,), jnp.float32)

    def zbody(j, _):
        pltpu.sync_copy(
            rows.at[0, pl.ds(0, ZERO_CHUNK), :],
            acc.at[pl.ds(s * TILE_ACC + j * ZERO_CHUNK, ZERO_CHUNK), :],
        )
        return 0

    lax.fori_loop(0, TILE_ACC // ZERO_CHUNK, zbody, 0)
    if with_rowsum:
        @pl.when(c == 0)
        def _():
            zcol = rows.at[0, 0, pl.ds(0, 16)]
            n_z1_chunks = N_ROWS // 16
            n_z1 = n_z1_chunks // NUM_TILES + jnp.where(
                s < n_z1_chunks % NUM_TILES, 1, 0)

            def z1body(j, _):
                pltpu.sync_copy(zcol, acc1.at[pl.ds((s + j * NUM_TILES) * 16, 16)])
                return 0

            lax.fori_loop(0, n_z1, z1body, 0)

    plsc.subcore_barrier()

    n_my = N_BLOCKS // NUM_TILES + jnp.where(s < N_BLOCKS % NUM_TILES, 1, 0)
    tab_c = table_hbm.at[c]

    def ebody(j, _):
        @pl.when(j > 0)
        def _():
            for k in range(BLK):
                pltpu.make_async_copy(tab_c.at[pl.ds(0, CHUNK), :],
                                      rows.at[k], sem_s).wait()
            if with_rowsum:
                @pl.when(c == 0)
                def _():
                    for k in range(BLK):
                        pltpu.make_async_copy(rs_hbm.at[pl.ds(0, CHUNK)],
                                              ones, sem_1).wait()

        b = (s + j * NUM_TILES) * BLK
        pltpu.sync_copy(src_hbm.at[pl.ds(b, BLK), :], idx_s)
        pltpu.sync_copy(dst_hbm.at[pl.ds(b, BLK), :], idx_d)
        gathers = [
            pltpu.async_copy(tab_c.at[idx_s.at[k]], rows.at[k], sem_g)
            for k in range(BLK)
        ]
        for k in range(BLK):
            gathers[k].wait()
            pltpu.async_copy(rows.at[k], acc.at[idx_d.at[k]], sem_s, add=True)
            if with_rowsum:
                @pl.when(c == 0)
                def _():
                    pltpu.async_copy(ones, acc1.at[idx_d.at[k]], sem_1, add=True)
        return 0

    lax.fori_loop(0, n_my, ebody, 0)
    for k in range(BLK):
        pltpu.make_async_copy(tab_c.at[pl.ds(0, CHUNK), :], rows.at[k], sem_s).wait()
    if with_rowsum:
        @pl.when(c == 0)
        def _():
            for k in range(BLK):
                pltpu.make_async_copy(rs_hbm.at[pl.ds(0, CHUNK)], ones, sem_1).wait()
    plsc.subcore_barrier()

    n_wb = N_WB // NUM_TILES + jnp.where(s < N_WB % NUM_TILES, 1, 0)

    def wbody(j, _):
        r0 = (s + j * NUM_TILES) * WB_CHUNK
        pltpu.sync_copy(acc.at[pl.ds(r0, WB_CHUNK), :],
                        out_hbm.at[c, pl.ds(r0, WB_CHUNK), :])
        if with_rowsum:
            @pl.when(c == 0)
            def _():
                pltpu.sync_copy(acc1.at[pl.ds(r0, WB_CHUNK)],
                                rs_hbm.at[pl.ds(r0, WB_CHUNK)])
        return 0

    lax.fori_loop(0, n_wb, wbody, 0)


def _make_spmm(with_rowsum):
    out_type = [jax.ShapeDtypeStruct((NUM_SC, N_ROWS, HDIM), jnp.float32)]
    scratch = [
        pltpu.VMEM((BLK, CHUNK), jnp.int32),
        pltpu.VMEM((BLK, CHUNK), jnp.int32),
        pltpu.VMEM((BLK, CHUNK, HDIM), jnp.float32),
        pltpu.VMEM((CHUNK,), jnp.float32),
        pltpu.VMEM_SHARED((N_ROWS, HDIM), jnp.float32),
    ]
    if with_rowsum:
        out_type.append(jax.ShapeDtypeStruct((N_ROWS,), jnp.float32))
        scratch.append(pltpu.VMEM_SHARED((N_ROWS,), jnp.float32))
    scratch += [pltpu.SemaphoreType.DMA] * 3
    return pl.kernel(
        functools.partial(_spmm_body, with_rowsum),
        out_type=tuple(out_type) if with_rowsum else out_type[0],
        mesh=_SC_MESH,
        scratch_types=scratch,
        compiler_params=_SC_PARAMS,
        name="spmm_rowsum" if with_rowsum else "spmm",
    )


_spmm_first = _make_spmm(True)
_spmm = _make_spmm(False)

_GB = BATCH_SIZE // NUM_TILES


def _gather_body(t2, i2, rs, u_idx, i_idx, u_out, i_out, rsg_out,
                 idx_v, rows_v, val_v, sem):
    c = lax.axis_index("c")
    s = lax.axis_index("s")
    base = s * _GB

    for half in range(_GB // CHUNK):
        o = base + half * CHUNK
        pltpu.sync_copy(u_idx.at[pl.ds(o, CHUNK)], idx_v)
        pltpu.async_copy(t2.at[c].at[idx_v], rows_v, sem).wait()
        pltpu.sync_copy(rows_v, u_out.at[c, pl.ds(o, CHUNK), :])

        @pl.when(c == 0)
        def _():
            pltpu.async_copy(rs.at[idx_v], val_v, sem).wait()
            pltpu.sync_copy(val_v, rsg_out.at[pl.ds(o, CHUNK)])

        pltpu.sync_copy(i_idx.at[pl.ds(o, CHUNK)], idx_v)
        pltpu.async_copy(i2.at[c].at[idx_v], rows_v, sem).wait()
        pltpu.sync_copy(rows_v, i_out.at[c, pl.ds(o, CHUNK), :])


_gather = pl.kernel(
    _gather_body,
    out_type=(
        jax.ShapeDtypeStruct((NUM_SC, BATCH_SIZE, HDIM), jnp.float32),
        jax.ShapeDtypeStruct((NUM_SC, BATCH_SIZE, HDIM), jnp.float32),
        jax.ShapeDtypeStruct((BATCH_SIZE,), jnp.float32),
    ),
    mesh=_SC_MESH,
    scratch_types=[
        pltpu.VMEM((CHUNK,), jnp.int32),
        pltpu.VMEM((CHUNK, HDIM), jnp.float32),
        pltpu.VMEM((CHUNK,), jnp.float32),
        pltpu.SemaphoreType.DMA,
    ],
    compiler_params=_SC_PARAMS,
    name="batch_gather",
)


_SCALE_BLK = 2000


def _split_kernel(t_ref, o_ref):
    o_ref[...] = t_ref[...].reshape(_SCALE_BLK, NUM_SC, HDIM).transpose(1, 0, 2)


def _split_planes(t):
    grid = N_ROWS // _SCALE_BLK
    return pl.pallas_call(
        _split_kernel,
        grid=(grid,),
        in_specs=[pl.BlockSpec((_SCALE_BLK, DIM), lambda i: (i, 0))],
        out_specs=pl.BlockSpec((NUM_SC, _SCALE_BLK, HDIM), lambda i: (0, i, 0)),
        out_shape=jax.ShapeDtypeStruct((NUM_SC, N_ROWS, HDIM), jnp.float32),
    )(t)


def _scale_kernel(t_ref, rs_ref, o_ref):
    rs = rs_ref[...]
    inv = jnp.where(rs > 0, 1.0 / rs, 0.0)
    o_ref[...] = t_ref[...] * (inv * inv)[None]


def _scale_rows(t, rs):
    grid = N_ROWS // _SCALE_BLK
    return pl.pallas_call(
        _scale_kernel,
        grid=(grid,),
        in_specs=[
            pl.BlockSpec((NUM_SC, _SCALE_BLK, HDIM), lambda i: (0, i, 0)),
            pl.BlockSpec((_SCALE_BLK, 1), lambda i: (i, 0)),
        ],
        out_specs=pl.BlockSpec((NUM_SC, _SCALE_BLK, HDIM), lambda i: (0, i, 0)),
        out_shape=jax.ShapeDtypeStruct((NUM_SC, N_ROWS, HDIM), jnp.float32),
    )(t, rs.reshape(N_ROWS, 1))


def _score_kernel(u_ref, i_ref, rs_ref, o_ref):
    dot = jnp.sum(u_ref[0] * i_ref[0], axis=1, keepdims=True)
    dot += jnp.sum(u_ref[1] * i_ref[1], axis=1, keepdims=True)
    rs = rs_ref[...]
    inv = jnp.where(rs > 0, 1.0 / rs, 0.0)
    o_ref[...] = jax.nn.sigmoid(dot * inv)


def _score(u_rows, i_rows, rs_g):
    out = pl.pallas_call(
        _score_kernel,
        out_shape=jax.ShapeDtypeStruct((BATCH_SIZE, 1), jnp.float32),
    )(u_rows, i_rows, rs_g.reshape(BATCH_SIZE, 1))
    return out.reshape(BATCH_SIZE)


def kernel(user_indices, item_indices, user_table, item_table, adj_rows, adj_cols):
    del user_table
    src_r = adj_rows.reshape(N_IDX_ROWS, CHUNK)
    src_c = adj_cols.reshape(N_IDX_ROWS, CHUNK)
    it0 = _split_planes(item_table)
    t1, rowsum = _spmm_first(src_c, src_r, it0)
    us1 = _scale_rows(t1, rowsum)
    i1 = _spmm(src_r, src_c, us1)
    t2 = _spmm(src_c, src_r, i1)
    us2 = _scale_rows(t2, rowsum)
    i2 = _spmm(src_r, src_c, us2)
    u_rows, i_rows, rs_g = _gather(t2, i2, rowsum, user_indices, item_indices)
    return _score(u_rows, i_rows, rs_g)
